# Initial kernel scaffold; baseline (speedup 1.0000x reference)
import jax
import jax.numpy as jnp
from jax.experimental import pallas as pl


def kernel(x, Wr, Wg, Wu, Wd, Wsg, Wsu, Wsd):
    raise NotImplementedError("write your pallas kernel here")


# speedup vs baseline: 1.6626x; 1.6626x over previous
import functools

import jax
import jax.numpy as jnp
from jax.experimental import pallas as pl
from jax.experimental.pallas import tpu as pltpu

N, D, H, E, K = 2048, 768, 2048, 8, 2


def _router_body(x_ref, wr_ref, ti_ref, tw_ref, aux_ref):
    x = x_ref[...]
    logits = jax.lax.dot_general(
        x, wr_ref[...], (((1,), (1,)), ((), ())),
        preferred_element_type=jnp.float32)
    iota = jax.lax.broadcasted_iota(jnp.int32, (N, E), 1)
    m1 = jnp.max(logits, axis=1, keepdims=True)
    i1 = jnp.min(jnp.where(logits == m1, iota, E), axis=1, keepdims=True)
    l2 = jnp.where(iota == i1, -jnp.inf, logits)
    m2 = jnp.max(l2, axis=1, keepdims=True)
    i2 = jnp.min(jnp.where(l2 == m2, iota, E), axis=1, keepdims=True)
    b = jnp.exp(m2 - m1)
    w1 = 1.0 / (1.0 + b)
    w2 = b / (1.0 + b)
    ti_ref[...] = jnp.concatenate([i1, i2], axis=1)
    tw_ref[...] = jnp.concatenate([w1, w2], axis=1)
    p = jnp.exp(logits - m1)
    p = p / jnp.sum(p, axis=1, keepdims=True)
    p_mean = jnp.sum(p, axis=0) / N
    counts = jnp.sum(
        (iota == i1).astype(jnp.float32) + (iota == i2).astype(jnp.float32),
        axis=0)
    f = counts / (N * K)
    aux_ref[...] = (E * jnp.sum(f * p_mean)).reshape(1, 1)


def _router(xf, Wr):
    return pl.pallas_call(
        _router_body,
        out_shape=[
            jax.ShapeDtypeStruct((N, K), jnp.int32),
            jax.ShapeDtypeStruct((N, K), jnp.float32),
            jax.ShapeDtypeStruct((1, 1), jnp.float32),
        ],
        in_specs=[
            pl.BlockSpec((N, D), lambda: (0, 0)),
            pl.BlockSpec((E, D), lambda: (0, 0)),
        ],
        out_specs=[
            pl.BlockSpec((N, K), lambda: (0, 0)),
            pl.BlockSpec((N, K), lambda: (0, 0)),
            pl.BlockSpec((1, 1), lambda: (0, 0)),
        ],
    )(xf, Wr)


def _silu(z):
    return z * jax.nn.sigmoid(z)


def _dense_body(x_ref, ti_ref, tw_ref, wg_ref, wu_ref, wd_ref, out_ref):
    e = pl.program_id(1)
    x = x_ref[...]
    h = _silu(jax.lax.dot_general(
        x, wg_ref[0], (((1,), (1,)), ((), ())),
        preferred_element_type=jnp.float32))
    h = h * jax.lax.dot_general(
        x, wu_ref[0], (((1,), (1,)), ((), ())),
        preferred_element_type=jnp.float32)
    y = jax.lax.dot_general(
        h, wd_ref[0], (((1,), (1,)), ((), ())),
        preferred_element_type=jnp.float32)
    combine = jnp.sum(
        jnp.where(ti_ref[...] == e, tw_ref[...], 0.0), axis=1, keepdims=True)
    contrib = combine * y

    @pl.when(e == 0)
    def _():
        out_ref[...] = contrib

    @pl.when(e != 0)
    def _():
        out_ref[...] += contrib


def _dense_moe(xf, ti, tw, Wg, Wu, Wd):
    TT = 4
    bm = N // TT
    return pl.pallas_call(
        _dense_body,
        grid=(TT, E),
        out_shape=jax.ShapeDtypeStruct((N, D), jnp.float32),
        in_specs=[
            pl.BlockSpec((bm, D), lambda t, e: (t, 0)),
            pl.BlockSpec((bm, K), lambda t, e: (t, 0)),
            pl.BlockSpec((bm, K), lambda t, e: (t, 0)),
            pl.BlockSpec((1, H, D), lambda t, e: (e, 0, 0)),
            pl.BlockSpec((1, H, D), lambda t, e: (e, 0, 0)),
            pl.BlockSpec((1, D, H), lambda t, e: (e, 0, 0)),
        ],
        out_specs=pl.BlockSpec((bm, D), lambda t, e: (t, 0)),
    )(xf, ti, tw, Wg, Wu, Wd)


def _shared_body(x_ref, prev_ref, wsg_ref, wsu_ref, wsd_ref, out_ref):
    x = x_ref[...]
    hs = _silu(jax.lax.dot_general(
        x, wsg_ref[...], (((1,), (1,)), ((), ())),
        preferred_element_type=jnp.float32))
    hs = hs * jax.lax.dot_general(
        x, wsu_ref[...], (((1,), (1,)), ((), ())),
        preferred_element_type=jnp.float32)
    ys = jax.lax.dot_general(
        hs, wsd_ref[...], (((1,), (1,)), ((), ())),
        preferred_element_type=jnp.float32)
    out_ref[...] = ys + prev_ref[...]


def _shared_add(xf, prev, Wsg, Wsu, Wsd):
    TT = 4
    bm = N // TT
    return pl.pallas_call(
        _shared_body,
        grid=(TT,),
        out_shape=jax.ShapeDtypeStruct((N, D), jnp.float32),
        in_specs=[
            pl.BlockSpec((bm, D), lambda t: (t, 0)),
            pl.BlockSpec((bm, D), lambda t: (t, 0)),
            pl.BlockSpec((H, D), lambda t: (0, 0)),
            pl.BlockSpec((H, D), lambda t: (0, 0)),
            pl.BlockSpec((D, H), lambda t: (0, 0)),
        ],
        out_specs=pl.BlockSpec((bm, D), lambda t: (t, 0)),
    )(xf, prev, Wsg, Wsu, Wsd)


def kernel(x, Wr, Wg, Wu, Wd, Wsg, Wsu, Wsd):
    Bz, T_, C = x.shape
    xf = x.reshape(-1, C)
    ti, tw, aux = _router(xf, Wr)
    oute = _dense_moe(xf, ti, tw, Wg, Wu, Wd)
    out = _shared_add(xf, oute, Wsg, Wsu, Wsd)
    return out.reshape(Bz, T_, C), aux[0, 0]
